# baseline (speedup 1.0000x reference)
import jax
import jax.numpy as jnp
from jax.experimental import pallas as pl
from jax.experimental.pallas import tpu as pltpu

_N_E = 1024
_E_DIM = 64
_BETA = 0.25
_TOK = 16384
_TILE = 1024
_GRID = _TOK // _TILE


def _vq_body(z_ref, emb_ref, esq_ref, zsq_ref, loss_ref, zq_ref, enc_ref, idx_ref):
    i = pl.program_id(0)
    zr = z_ref[...].reshape(_E_DIM, _TILE)
    zf = zr.T
    emb = emb_ref[...]
    esq = esq_ref[...]
    zsq = zsq_ref[...]

    mm = jax.lax.dot_general(
        zf, emb, (((1,), (1,)), ((), ())),
        preferred_element_type=jnp.float32)
    d = zsq + esq - 2.0 * mm

    dmin = jnp.min(d, axis=1, keepdims=True)
    col = jax.lax.broadcasted_iota(jnp.int32, d.shape, 1)
    idx = jnp.min(jnp.where(d == dmin, col, _N_E), axis=1)

    onehot = (col == idx[:, None]).astype(jnp.float32)
    enc_ref[...] = onehot
    idx_ref[...] = idx[:, None]

    zq = jax.lax.dot_general(
        onehot, emb, (((1,), (0,)), ((), ())),
        preferred_element_type=jnp.float32)
    zq_st = zf + (zq - zf)
    zq_ref[...] = zq_st.T.reshape(1, _E_DIM, 32, 32)

    diff = zf - zq
    part = jnp.sum(diff * diff, keepdims=True)

    @pl.when(i == 0)
    def _init():
        loss_ref[...] = jnp.zeros((1, 1), jnp.float32)

    loss_ref[...] += part

    @pl.when(i == _GRID - 1)
    def _fin():
        loss_ref[...] = loss_ref[...] * ((1.0 + _BETA) / (_TOK * _E_DIM))


def _vq_call(z, emb_weight, esq, zsq):
    return pl.pallas_call(
        _vq_body,
        grid=(_GRID,),
        in_specs=[
            pl.BlockSpec((1, _E_DIM, 32, 32), lambda i: (i, 0, 0, 0)),
            pl.BlockSpec((_N_E, _E_DIM), lambda i: (0, 0)),
            pl.BlockSpec((1, _N_E), lambda i: (0, 0)),
            pl.BlockSpec((_TILE, 1), lambda i: (i, 0)),
        ],
        out_specs=[
            pl.BlockSpec((1, 1), lambda i: (0, 0)),
            pl.BlockSpec((1, _E_DIM, 32, 32), lambda i: (i, 0, 0, 0)),
            pl.BlockSpec((_TILE, _N_E), lambda i: (i, 0)),
            pl.BlockSpec((_TILE, 1), lambda i: (i, 0)),
        ],
        out_shape=[
            jax.ShapeDtypeStruct((1, 1), jnp.float32),
            jax.ShapeDtypeStruct((16, _E_DIM, 32, 32), jnp.float32),
            jax.ShapeDtypeStruct((_TOK, _N_E), jnp.float32),
            jax.ShapeDtypeStruct((_TOK, 1), jnp.int32),
        ],
        compiler_params=pltpu.CompilerParams(
            dimension_semantics=("arbitrary",)),
    )(z, emb_weight, esq, zsq)


def kernel(z, emb_weight):
    zp = jnp.transpose(z, (0, 2, 3, 1))
    zf = zp.reshape(-1, _E_DIM)
    zsq = jnp.sum(zf ** 2, axis=1, keepdims=True)
    esq = jnp.sum(emb_weight ** 2, axis=1)[None, :]
    loss2, z_q, enc, idx = _vq_call(z, emb_weight, esq, zsq)
    return (loss2[0, 0], z_q, enc, idx)


# speedup vs baseline: 1.0602x; 1.0602x over previous
import jax
import jax.numpy as jnp
from jax.experimental import pallas as pl
from jax.experimental.pallas import tpu as pltpu

_N_E = 1024
_E_DIM = 64
_BETA = 0.25
_TOK = 16384
_TILE = 1024
_GRID = _TOK // _TILE


def _vq_body(z_ref, emb_ref, esq_ref, zsq_ref, loss_ref, zq_ref, enc_ref, idx_ref):
    i = pl.program_id(0)
    zr = z_ref[...].reshape(_E_DIM, _TILE)
    emb = emb_ref[...]
    esq = esq_ref[...]
    zsq = zsq_ref[...]

    mm = jax.lax.dot_general(
        zr, emb, (((0,), (1,)), ((), ())),
        preferred_element_type=jnp.float32)
    d = zsq + esq - 2.0 * mm

    dmin = jnp.min(d, axis=1, keepdims=True)
    col = jax.lax.broadcasted_iota(jnp.int32, d.shape, 1)
    idx = jnp.min(jnp.where(d == dmin, col, _N_E), axis=1)

    onehot = (col == idx[:, None]).astype(jnp.float32)
    enc_ref[...] = onehot
    idx_ref[...] = idx[:, None]

    zqt = jax.lax.dot_general(
        emb, onehot, (((0,), (1,)), ((), ())),
        preferred_element_type=jnp.float32)
    zq_st = zr + (zqt - zr)
    zq_ref[...] = zq_st.reshape(1, _E_DIM, 32, 32)

    diff = zr - zqt
    part = jnp.sum(diff * diff, keepdims=True)

    @pl.when(i == 0)
    def _init():
        loss_ref[...] = jnp.zeros((1, 1), jnp.float32)

    loss_ref[...] += part

    @pl.when(i == _GRID - 1)
    def _fin():
        loss_ref[...] = loss_ref[...] * ((1.0 + _BETA) / (_TOK * _E_DIM))


def _vq_call(z, emb_weight, esq, zsq):
    return pl.pallas_call(
        _vq_body,
        grid=(_GRID,),
        in_specs=[
            pl.BlockSpec((1, _E_DIM, 32, 32), lambda i: (i, 0, 0, 0)),
            pl.BlockSpec((_N_E, _E_DIM), lambda i: (0, 0)),
            pl.BlockSpec((1, _N_E), lambda i: (0, 0)),
            pl.BlockSpec((_TILE, 1), lambda i: (i, 0)),
        ],
        out_specs=[
            pl.BlockSpec((1, 1), lambda i: (0, 0)),
            pl.BlockSpec((1, _E_DIM, 32, 32), lambda i: (i, 0, 0, 0)),
            pl.BlockSpec((_TILE, _N_E), lambda i: (i, 0)),
            pl.BlockSpec((_TILE, 1), lambda i: (i, 0)),
        ],
        out_shape=[
            jax.ShapeDtypeStruct((1, 1), jnp.float32),
            jax.ShapeDtypeStruct((16, _E_DIM, 32, 32), jnp.float32),
            jax.ShapeDtypeStruct((_TOK, _N_E), jnp.float32),
            jax.ShapeDtypeStruct((_TOK, 1), jnp.int32),
        ],
        compiler_params=pltpu.CompilerParams(
            dimension_semantics=("arbitrary",)),
    )(z, emb_weight, esq, zsq)


def kernel(z, emb_weight):
    zp = jnp.transpose(z, (0, 2, 3, 1))
    zf = zp.reshape(-1, _E_DIM)
    zsq = jnp.sum(zf ** 2, axis=1, keepdims=True)
    esq = jnp.sum(emb_weight ** 2, axis=1)[None, :]
    loss2, z_q, enc, idx = _vq_call(z, emb_weight, esq, zsq)
    return (loss2[0, 0], z_q, enc, idx)


# speedup vs baseline: 1.4304x; 1.3492x over previous
import jax
import jax.numpy as jnp
from jax.experimental import pallas as pl
from jax.experimental.pallas import tpu as pltpu

_N_E = 1024
_E_DIM = 64
_BETA = 0.25
_TOK = 16384
_TILE = 1024
_GRID = _TOK // _TILE


def _vq_body(z_ref, emb_ref, esq_ref, zsq_ref, loss_ref, zq_ref, enc_ref, idx_ref):
    i = pl.program_id(0)
    zr = z_ref[...].reshape(_E_DIM, _TILE)
    emb = emb_ref[...]
    esq = esq_ref[...]
    zsq = zsq_ref[...]

    mm = jax.lax.dot_general(
        zr, emb, (((0,), (1,)), ((), ())),
        preferred_element_type=jnp.float32)
    d = zsq + esq - 2.0 * mm

    dmin = jnp.min(d, axis=1, keepdims=True)
    col = jax.lax.broadcasted_iota(jnp.int32, d.shape, 1)
    idx = jnp.min(jnp.where(d == dmin, col, _N_E), axis=1)

    onehot = (col == idx[:, None]).astype(jnp.float32)
    enc_ref[...] = onehot
    idx_ref[...] = idx[:, None]

    zqt = jax.lax.dot_general(
        emb, onehot, (((0,), (1,)), ((), ())),
        preferred_element_type=jnp.float32)
    zq_st = zr + (zqt - zr)
    zq_ref[...] = zq_st.reshape(1, _E_DIM, _TILE)

    diff = zr - zqt
    part = jnp.sum(diff * diff, keepdims=True)

    @pl.when(i == 0)
    def _init():
        loss_ref[...] = jnp.zeros((1, 1), jnp.float32)

    loss_ref[...] += part

    @pl.when(i == _GRID - 1)
    def _fin():
        loss_ref[...] = loss_ref[...] * ((1.0 + _BETA) / (_TOK * _E_DIM))


def _vq_call(z, emb_weight, esq, zsq):
    return pl.pallas_call(
        _vq_body,
        grid=(_GRID,),
        in_specs=[
            pl.BlockSpec((1, _E_DIM, _TILE), lambda i: (i, 0, 0)),
            pl.BlockSpec((_N_E, _E_DIM), lambda i: (0, 0)),
            pl.BlockSpec((1, _N_E), lambda i: (0, 0)),
            pl.BlockSpec((_TILE, 1), lambda i: (i, 0)),
        ],
        out_specs=[
            pl.BlockSpec((1, 1), lambda i: (0, 0)),
            pl.BlockSpec((1, _E_DIM, _TILE), lambda i: (i, 0, 0)),
            pl.BlockSpec((_TILE, _N_E), lambda i: (i, 0)),
            pl.BlockSpec((_TILE, 1), lambda i: (i, 0)),
        ],
        out_shape=[
            jax.ShapeDtypeStruct((1, 1), jnp.float32),
            jax.ShapeDtypeStruct((16, _E_DIM, _TILE), jnp.float32),
            jax.ShapeDtypeStruct((_TOK, _N_E), jnp.float32),
            jax.ShapeDtypeStruct((_TOK, 1), jnp.int32),
        ],
        compiler_params=pltpu.CompilerParams(
            dimension_semantics=("arbitrary",)),
    )(z.reshape(16, _E_DIM, _TILE), emb_weight, esq, zsq)


def kernel(z, emb_weight):
    zp = jnp.transpose(z, (0, 2, 3, 1))
    zf = zp.reshape(-1, _E_DIM)
    zsq = jnp.sum(zf ** 2, axis=1, keepdims=True)
    esq = jnp.sum(emb_weight ** 2, axis=1)[None, :]
    loss2, zq3, enc, idx = _vq_call(z, emb_weight, esq, zsq)
    return (loss2[0, 0], zq3.reshape(z.shape), enc, idx)
